# baseline (speedup 1.0000x reference)
import jax
import jax.numpy as jnp
from jax import lax
from jax.experimental import pallas as pl
from jax.experimental.pallas import tpu as pltpu
from jax.experimental.pallas import tpu_sc as plsc

_W = 512
NC, NS, L = 2, 16, 16
NW = NC * NS
B = 1048576
BPW = B // NW
CHUNK = 2048
NIDX = 128
NB = CHUNK // NIDX
NCH = BPW // CHUNK
D = 8
T = _W * _W
TPW = T // NW

_SC_PARAMS = pltpu.CompilerParams(
    needs_layout_passes=False, use_tc_tiling_on_sc=False
)
_MESH = plsc.VectorSubcoreMesh(core_axis_name="c", subcore_axis_name="s")


def _pad_body(img_hbm, table_hbm, buf):
    wid = lax.axis_index("s") * NC + lax.axis_index("c")
    base = wid * TPW
    pltpu.sync_copy(img_hbm.at[pl.ds(base, TPW)], buf.at[:, pl.ds(0, 3)])
    pltpu.sync_copy(buf, table_hbm.at[pl.ds(base, TPW)])


def _coords_copy(coords_hbm, coords_v, sem_c, t, base, slot):
    off = base + t * CHUNK
    return pltpu.make_async_copy(
        coords_hbm.at[pl.ds(2 * off, 2 * CHUNK)], coords_v.at[slot], sem_c.at[slot]
    )


def _body(coords_hbm, table_hbm, out_hbm, coords_v, idx_v, rows_v, sem_c, sem_g):
    wid = lax.axis_index("s") * NC + lax.axis_index("c")
    base = wid * BPW
    lane2 = lax.iota(jnp.int32, L) * 2

    _coords_copy(coords_hbm, coords_v, sem_c, 0, base, 0).start()

    def pair_body(tt, carry):
        for slot in range(2):
            t = 2 * tt + slot
            off = base + t * CHUNK
            _coords_copy(coords_hbm, coords_v, sem_c, t, base, slot).wait()

            @pl.when(t + 1 < NCH)
            def _():
                _coords_copy(
                    coords_hbm, coords_v, sem_c, t + 1, base, 1 - slot
                ).start()

            def idx_body(k, c2):
                for jj in range(NIDX // L):
                    pos = k * (2 * NIDX) + jj * (2 * L) + lane2
                    r = plsc.load_gather(coords_v.at[slot], [pos])
                    c = plsc.load_gather(coords_v.at[slot], [pos + 1])
                    idx_v[k, pl.ds(jj * L, L)] = r * _W + c
                return c2

            lax.fori_loop(0, NB, idx_body, 0)

            descs = [
                pltpu.async_copy(
                    table_hbm.at[idx_v.at[k]],
                    rows_v.at[pl.ds(k * NIDX, NIDX)],
                    sem_g,
                )
                for k in range(NB)
            ]
            for d in descs:
                d.wait()
            pltpu.sync_copy(
                rows_v.at[:, pl.ds(0, 3)], out_hbm.at[pl.ds(off, CHUNK)]
            )
        return carry

    lax.fori_loop(0, NCH // 2, pair_body, 0)


def kernel(coords, image):
    coords_flat = coords.reshape(-1).astype(jnp.int32)
    img_rows = image.reshape(T, 3)

    pad_f = pl.kernel(
        _pad_body,
        out_type=jax.ShapeDtypeStruct((T, D), jnp.float32),
        mesh=_MESH,
        compiler_params=_SC_PARAMS,
        scratch_types=[pltpu.VMEM((TPW, D), jnp.float32)],
    )
    table = pad_f(img_rows)

    f = pl.kernel(
        _body,
        out_type=jax.ShapeDtypeStruct((B, 3), jnp.float32),
        mesh=_MESH,
        compiler_params=_SC_PARAMS,
        scratch_types=[
            pltpu.VMEM((2, 2 * CHUNK), jnp.int32),
            pltpu.VMEM((NB, NIDX), jnp.int32),
            pltpu.VMEM((CHUNK, D), jnp.float32),
            pltpu.SemaphoreType.DMA((2,)),
            pltpu.SemaphoreType.DMA,
        ],
    )
    return f(coords_flat, table)


# speedup vs baseline: 1.8168x; 1.8168x over previous
import jax
import jax.numpy as jnp
from jax import lax
from jax.experimental import pallas as pl
from jax.experimental.pallas import tpu as pltpu
from jax.experimental.pallas import tpu_sc as plsc

_W = 512
NC, NS, L = 2, 16, 16
NW = NC * NS
B = 1048576
BPW = B // NW
CHUNK = 2048
NIDX = 128
NB = CHUNK // NIDX
NCH = BPW // CHUNK
D = 8
T = _W * _W
TPW = T // NW

_SC_PARAMS = pltpu.CompilerParams(
    needs_layout_passes=False, use_tc_tiling_on_sc=False
)
_MESH = plsc.VectorSubcoreMesh(core_axis_name="c", subcore_axis_name="s")

def _lane_pats():
    lane = lax.iota(jnp.int32, L)
    pix = [(lane + 16 * j) // 3 for j in range(3)]
    chan = [(lane + 16 * j) % 3 for j in range(3)]
    return pix, chan


def _pad_body(img_hbm, table_hbm, buf3, buf8):
    wid = lax.axis_index("s") * NC + lax.axis_index("c")
    base = wid * TPW
    pltpu.sync_copy(img_hbm.at[pl.ds(3 * base, 3 * TPW)], buf3)
    pix_pats, chan_pats = _lane_pats()

    def expand_body(g, c2):
        for j in range(3):
            v = buf3[pl.ds(48 * g + 16 * j, L)]
            plsc.store_scatter(buf8, [pix_pats[j] + 16 * g, chan_pats[j]], v)
        return c2

    lax.fori_loop(0, TPW // 16, expand_body, 0)
    pltpu.sync_copy(buf8, table_hbm.at[pl.ds(base, TPW)])


def _coords_copy(coords_hbm, coords_v, sem_c, t, base):
    off = base + t * CHUNK
    return pltpu.make_async_copy(
        coords_hbm.at[pl.ds(2 * off, 2 * CHUNK)], coords_v, sem_c
    )


def _gather_descs(table_hbm, idx_v, rows_v, sem_g):
    return [
        pltpu.make_async_copy(
            table_hbm.at[idx_v.at[k]],
            rows_v.at[pl.ds(k * NIDX, NIDX)],
            sem_g,
        )
        for k in range(NB)
    ]


def _body(
    coords_hbm, table_hbm, out_hbm,
    coords_v0, coords_v1, idx_v0, idx_v1, rows_v0, rows_v1, eout_v,
    sem_c0, sem_c1, sem_g0, sem_g1,
):
    wid = lax.axis_index("s") * NC + lax.axis_index("c")
    base = wid * BPW
    lane2 = lax.iota(jnp.int32, L) * 2
    coords_bufs = [coords_v0, coords_v1]
    idx_bufs = [idx_v0, idx_v1]
    rows_bufs = [rows_v0, rows_v1]
    sem_cs = [sem_c0, sem_c1]
    sem_gs = [sem_g0, sem_g1]
    pix_pats, chan_pats = _lane_pats()

    def stage_coords(t, slot):
        return _coords_copy(coords_hbm, coords_bufs[slot], sem_cs[slot], t, base)

    def compute_idx(slot):
        coords_v, idx_v = coords_bufs[slot], idx_bufs[slot]

        def idx_body(k, c2):
            for jj in range(NIDX // L):
                pos = k * (2 * NIDX) + jj * (2 * L) + lane2
                r = plsc.load_gather(coords_v, [pos])
                c = plsc.load_gather(coords_v, [pos + 1])
                idx_v[k, pl.ds(jj * L, L)] = r * _W + c
            return c2

        lax.fori_loop(0, NB, idx_body, 0)

    def compress_and_out(slot, t_prev):
        rows_v = rows_bufs[slot]
        off = base + t_prev * CHUNK

        def cb(g, c2):
            for j in range(3):
                v = plsc.load_gather(rows_v, [pix_pats[j] + 16 * g, chan_pats[j]])
                eout_v[pl.ds(48 * g + 16 * j, L)] = v
            return c2

        lax.fori_loop(0, CHUNK // 16, cb, 0)
        pltpu.sync_copy(eout_v, out_hbm.at[pl.ds(3 * off, 3 * CHUNK)])

    stage_coords(0, 0).start()

    def pair_body(tt, carry):
        for slot in range(2):
            t = 2 * tt + slot
            stage_coords(t, slot).wait()

            @pl.when(t + 1 < NCH)
            def _():
                stage_coords(t + 1, 1 - slot).start()

            compute_idx(slot)
            for d in _gather_descs(table_hbm, idx_bufs[slot], rows_bufs[slot], sem_gs[slot]):
                d.start()

            @pl.when(t > 0)
            def _():
                for d in _gather_descs(
                    table_hbm, idx_bufs[1 - slot], rows_bufs[1 - slot], sem_gs[1 - slot]
                ):
                    d.wait()
                compress_and_out(1 - slot, t - 1)
        return carry

    lax.fori_loop(0, NCH // 2, pair_body, 0)
    for d in _gather_descs(table_hbm, idx_bufs[1], rows_bufs[1], sem_gs[1]):
        d.wait()
    compress_and_out(1, NCH - 1)


def kernel(coords, image):
    coords_flat = coords.reshape(-1).astype(jnp.int32)
    img_flat = image.reshape(-1)

    pad_f = pl.kernel(
        _pad_body,
        out_type=jax.ShapeDtypeStruct((T, D), jnp.float32),
        mesh=_MESH,
        compiler_params=_SC_PARAMS,
        scratch_types=[
            pltpu.VMEM((3 * TPW,), jnp.float32),
            pltpu.VMEM((TPW, D), jnp.float32),
        ],
    )
    table = pad_f(img_flat)

    f = pl.kernel(
        _body,
        out_type=jax.ShapeDtypeStruct((3 * B,), jnp.float32),
        mesh=_MESH,
        compiler_params=_SC_PARAMS,
        scratch_types=[
            pltpu.VMEM((2 * CHUNK,), jnp.int32),
            pltpu.VMEM((2 * CHUNK,), jnp.int32),
            pltpu.VMEM((NB, NIDX), jnp.int32),
            pltpu.VMEM((NB, NIDX), jnp.int32),
            pltpu.VMEM((CHUNK, D), jnp.float32),
            pltpu.VMEM((CHUNK, D), jnp.float32),
            pltpu.VMEM((3 * CHUNK,), jnp.float32),
            pltpu.SemaphoreType.DMA,
            pltpu.SemaphoreType.DMA,
            pltpu.SemaphoreType.DMA,
            pltpu.SemaphoreType.DMA,
        ],
    )
    return f(coords_flat, table).reshape(B, 3)


# speedup vs baseline: 14.8701x; 8.1847x over previous
import jax
import jax.numpy as jnp
from jax import lax
from jax.experimental import pallas as pl
from jax.experimental.pallas import tpu as pltpu
from jax.experimental.pallas import tpu_sc as plsc

_W = 512
NC, NS, L = 2, 16, 16
NW = NC * NS
B = 1048576
BPW = B // NW
CHUNK = 2048
NIDX = 128
NB = CHUNK // NIDX
NCH = BPW // CHUNK
D = 8
T = _W * _W
TPW = T // NW

_SC_PARAMS = pltpu.CompilerParams(
    needs_layout_passes=False, use_tc_tiling_on_sc=False
)
_MESH = plsc.VectorSubcoreMesh(core_axis_name="c", subcore_axis_name="s")


def _pad_body(img_hbm, table_hbm, buf3, buf8):
    wid = lax.axis_index("s") * NC + lax.axis_index("c")
    base = wid * TPW
    pltpu.sync_copy(img_hbm.at[pl.ds(3 * base, 3 * TPW)], buf3)
    lane = lax.iota(jnp.int32, L)
    pix_pats = [(lane + 16 * j) // 3 for j in range(3)]
    chan_pats = [(lane + 16 * j) % 3 for j in range(3)]

    def expand_body(g, c2):
        for j in range(3):
            v = buf3[pl.ds(48 * g + 16 * j, L)]
            plsc.store_scatter(buf8, [pix_pats[j] + 16 * g, chan_pats[j]], v)
        return c2

    lax.fori_loop(0, TPW // 16, expand_body, 0)
    pltpu.sync_copy(buf8, table_hbm.at[pl.ds(base, TPW)])


def _col_copy(col_hbm, col_v, sem, t, base):
    off = base + t * CHUNK
    return pltpu.make_async_copy(col_hbm.at[pl.ds(off, CHUNK)], col_v, sem)


def _gather_descs(table_hbm, idx_v, rows_v, sem_g):
    return [
        pltpu.make_async_copy(
            table_hbm.at[idx_v.at[k]],
            rows_v.at[pl.ds(k * NIDX, NIDX)],
            sem_g,
        )
        for k in range(NB)
    ]


def _body(
    r_hbm, c_hbm, table_hbm, outr_hbm, outg_hbm, outb_hbm,
    r_v0, r_v1, c_v0, c_v1, idx_v0, idx_v1, rows_v0, rows_v1, pl_v,
    sem_r0, sem_r1, sem_c0, sem_c1, sem_g0, sem_g1,
):
    wid = lax.axis_index("s") * NC + lax.axis_index("c")
    base = wid * BPW
    lane = lax.iota(jnp.int32, L)
    r_bufs, c_bufs = [r_v0, r_v1], [c_v0, c_v1]
    idx_bufs, rows_bufs = [idx_v0, idx_v1], [rows_v0, rows_v1]
    sem_rs, sem_cs = [sem_r0, sem_r1], [sem_c0, sem_c1]
    sem_gs = [sem_g0, sem_g1]
    outs = [outr_hbm, outg_hbm, outb_hbm]

    def stage(t, slot):
        return (
            _col_copy(r_hbm, r_bufs[slot], sem_rs[slot], t, base),
            _col_copy(c_hbm, c_bufs[slot], sem_cs[slot], t, base),
        )

    def compute_idx(slot):
        r_v, c_v, idx_v = r_bufs[slot], c_bufs[slot], idx_bufs[slot]

        def idx_body(k, c2):
            for jj in range(NIDX // L):
                p = k * NIDX + jj * L
                r = r_v[pl.ds(p, L)]
                c = c_v[pl.ds(p, L)]
                idx_v[k, pl.ds(jj * L, L)] = r * _W + c
            return c2

        lax.fori_loop(0, NB, idx_body, 0)

    def split_and_out(slot, t_prev):
        rows_v = rows_bufs[slot]
        off = base + t_prev * CHUNK
        for j in range(3):
            chan = lane * 0 + j

            def cb(g, c2):
                v = plsc.load_gather(rows_v, [lane + 16 * g, chan])
                pl_v[pl.ds(16 * g, L)] = v
                return c2

            lax.fori_loop(0, CHUNK // 16, cb, 0)
            pltpu.sync_copy(pl_v, outs[j].at[pl.ds(off, CHUNK)])

    for d in stage(0, 0):
        d.start()

    def pair_body(tt, carry):
        for slot in range(2):
            t = 2 * tt + slot
            for d in stage(t, slot):
                d.wait()

            @pl.when(t + 1 < NCH)
            def _():
                for d in stage(t + 1, 1 - slot):
                    d.start()

            compute_idx(slot)
            for d in _gather_descs(table_hbm, idx_bufs[slot], rows_bufs[slot], sem_gs[slot]):
                d.start()

            @pl.when(t > 0)
            def _():
                for d in _gather_descs(
                    table_hbm, idx_bufs[1 - slot], rows_bufs[1 - slot], sem_gs[1 - slot]
                ):
                    d.wait()
                split_and_out(1 - slot, t - 1)
        return carry

    lax.fori_loop(0, NCH // 2, pair_body, 0)
    for d in _gather_descs(table_hbm, idx_bufs[1], rows_bufs[1], sem_gs[1]):
        d.wait()
    split_and_out(1, NCH - 1)


def kernel(coords, image):
    img_flat = image.reshape(-1)

    pad_f = pl.kernel(
        _pad_body,
        out_type=jax.ShapeDtypeStruct((T, D), jnp.float32),
        mesh=_MESH,
        compiler_params=_SC_PARAMS,
        scratch_types=[
            pltpu.VMEM((3 * TPW,), jnp.float32),
            pltpu.VMEM((TPW, D), jnp.float32),
        ],
    )
    table = pad_f(img_flat)

    f = pl.kernel(
        _body,
        out_type=(
            jax.ShapeDtypeStruct((B,), jnp.float32),
            jax.ShapeDtypeStruct((B,), jnp.float32),
            jax.ShapeDtypeStruct((B,), jnp.float32),
        ),
        mesh=_MESH,
        compiler_params=_SC_PARAMS,
        scratch_types=[
            pltpu.VMEM((CHUNK,), jnp.int32),
            pltpu.VMEM((CHUNK,), jnp.int32),
            pltpu.VMEM((CHUNK,), jnp.int32),
            pltpu.VMEM((CHUNK,), jnp.int32),
            pltpu.VMEM((NB, NIDX), jnp.int32),
            pltpu.VMEM((NB, NIDX), jnp.int32),
            pltpu.VMEM((CHUNK, D), jnp.float32),
            pltpu.VMEM((CHUNK, D), jnp.float32),
            pltpu.VMEM((CHUNK,), jnp.float32),
            pltpu.SemaphoreType.DMA,
            pltpu.SemaphoreType.DMA,
            pltpu.SemaphoreType.DMA,
            pltpu.SemaphoreType.DMA,
            pltpu.SemaphoreType.DMA,
            pltpu.SemaphoreType.DMA,
        ],
    )
    cr, cg, cb = f(
        coords[:, 0].astype(jnp.int32), coords[:, 1].astype(jnp.int32), table
    )
    return jnp.stack([cr, cg, cb], axis=-1)
